# Initial kernel scaffold; baseline (speedup 1.0000x reference)
import functools

import jax
import jax.numpy as jnp
from jax import lax
from jax.experimental import pallas as pl
from jax.experimental.pallas import tpu as pltpu
from jax.experimental.pallas import tpu_sc as plsc

N = 10000
E = 320000
D = 128
D_EDGE = 16

NC, NS, L = 2, 16, 16
NW = NC * NS
EPW = E // NW
C = 80
NCHUNK = EPW // C
RPT = N // NS

BM = 2000
BE = 4000


def _edge_proj_body(a_ref, w1_ref, w2_ref, w3_ref, o1_ref, o2_ref, o3_ref):
    a = a_ref[...]
    o1_ref[...] = jnp.dot(a, w1_ref[...], preferred_element_type=jnp.float32)
    o2_ref[...] = jnp.dot(a, w2_ref[...], preferred_element_type=jnp.float32)
    o3_ref[...] = jnp.dot(a, w3_ref[...], preferred_element_type=jnp.float32)


def _edge_proj(edge_attr, w1, w2, w3):
    wspec = pl.BlockSpec((D_EDGE, D), lambda i: (0, 0))
    ospec = pl.BlockSpec((BE, D), lambda i: (i, 0))
    return pl.pallas_call(
        _edge_proj_body,
        grid=(E // BE,),
        in_specs=[pl.BlockSpec((BE, D_EDGE), lambda i: (i, 0)), wspec, wspec, wspec],
        out_specs=[ospec, ospec, ospec],
        out_shape=[jax.ShapeDtypeStruct((E, D), jnp.float32)] * 3,
    )(edge_attr, w1, w2, w3)


def _proj_body(h_ref, wk_ref, wq_ref, wv_ref, ws_ref,
               bk_ref, bq_ref, bv_ref, bb_ref,
               k_ref, qv_ref, s_ref):
    h = h_ref[...]
    k_ref[...] = jnp.dot(h, wk_ref[...], preferred_element_type=jnp.float32) + bk_ref[...]
    qv_ref[:, :D] = jnp.dot(h, wq_ref[...], preferred_element_type=jnp.float32) + bq_ref[...]
    qv_ref[:, D:] = jnp.dot(h, wv_ref[...], preferred_element_type=jnp.float32) + bv_ref[...]
    s_ref[...] = jnp.dot(h, ws_ref[...], preferred_element_type=jnp.float32) + bb_ref[...]


def _fused_proj_body(agg_ref, sp_ref, wk_ref, wq_ref, wv_ref, ws_ref,
                     bk_ref, bq_ref, bv_ref, bb_ref,
                     h_ref, k_ref, qv_ref, s_ref):
    z = agg_ref[0] + agg_ref[1] + sp_ref[...]
    h = jnp.where(z > 0, z, jnp.exp(jnp.minimum(z, 0.0)) - 1.0)
    h_ref[...] = h
    k_ref[...] = jnp.dot(h, wk_ref[...], preferred_element_type=jnp.float32) + bk_ref[...]
    qv_ref[:, :D] = jnp.dot(h, wq_ref[...], preferred_element_type=jnp.float32) + bq_ref[...]
    qv_ref[:, D:] = jnp.dot(h, wv_ref[...], preferred_element_type=jnp.float32) + bv_ref[...]
    s_ref[...] = jnp.dot(h, ws_ref[...], preferred_element_type=jnp.float32) + bb_ref[...]


_WSPEC = pl.BlockSpec((D, D), lambda i: (0, 0))
_BSPEC = pl.BlockSpec((1, D), lambda i: (0, 0))
_HSPEC = pl.BlockSpec((BM, D), lambda i: (i, 0))
_QVSPEC = pl.BlockSpec((BM, 2 * D), lambda i: (i, 0))
_AGGSPEC = pl.BlockSpec((2, BM, D), lambda i: (0, i, 0))

_NODE_OUT = [
    jax.ShapeDtypeStruct((N, D), jnp.float32),
    jax.ShapeDtypeStruct((N, 2 * D), jnp.float32),
    jax.ShapeDtypeStruct((N, D), jnp.float32),
]


def _node_proj(x, wk, wq, wv, ws, bk, bq, bv, bb):
    return pl.pallas_call(
        _proj_body,
        grid=(N // BM,),
        in_specs=[_HSPEC] + [_WSPEC] * 4 + [_BSPEC] * 4,
        out_specs=[_HSPEC, _QVSPEC, _HSPEC],
        out_shape=_NODE_OUT,
    )(x, wk, wq, wv, ws, bk, bq, bv, bb)


def _fused_proj(agg, sp, wk, wq, wv, ws, bk, bq, bv, bb):
    return pl.pallas_call(
        _fused_proj_body,
        grid=(N // BM,),
        in_specs=[_AGGSPEC, _HSPEC] + [_WSPEC] * 4 + [_BSPEC] * 4,
        out_specs=[_HSPEC, _HSPEC, _QVSPEC, _HSPEC],
        out_shape=[jax.ShapeDtypeStruct((N, D), jnp.float32)] + _NODE_OUT,
    )(agg, sp, wk, wq, wv, ws, bk, bq, bv, bb)


def _final_body(agg_ref, sp_ref, wl_ref, bl_ref, lin_ref, sig_ref):
    z = agg_ref[0] + agg_ref[1] + sp_ref[...]
    h = jnp.where(z > 0, z, jnp.exp(jnp.minimum(z, 0.0)) - 1.0)
    lin = jnp.dot(h, wl_ref[...], preferred_element_type=jnp.float32) + bl_ref[...]
    lin_ref[...] = lin
    sig_ref[...] = jax.nn.sigmoid(lin)


def _final(agg, sp, wl_pad, bl_pad):
    return pl.pallas_call(
        _final_body,
        grid=(N // BM,),
        in_specs=[_AGGSPEC, _HSPEC, _WSPEC, _BSPEC],
        out_specs=[_HSPEC, _HSPEC],
        out_shape=[jax.ShapeDtypeStruct((N, D), jnp.float32)] * 2,
    )(agg, sp, wl_pad, bl_pad)


def _sc_edge_body(k_hbm, qv_hbm, e_hbm, src_hbm, dst_hbm, z_hbm, out_hbm,
                  idx_s, idx_d, kbuf, qvbuf, ebuf, msgbuf, agg_sh,
                  sem_k, sem_qv, sem_e):
    c = lax.axis_index("c")
    s = lax.axis_index("s")
    wid = c * NS + s
    ebase = wid * EPW

    pltpu.sync_copy(z_hbm, agg_sh.at[pl.ds(s * RPT, RPT)])
    plsc.subcore_barrier()

    def chunk(ci, _):
        base = ebase + ci * C
        pltpu.sync_copy(src_hbm.at[pl.ds(base, C)], idx_s)
        pltpu.sync_copy(dst_hbm.at[pl.ds(base, C)], idx_d)
        cp_k = pltpu.async_copy(k_hbm.at[idx_d], kbuf, sem_k)
        cp_qv = pltpu.async_copy(qv_hbm.at[idx_s], qvbuf, sem_qv)
        cp_e = pltpu.async_copy(e_hbm.at[pl.ds(base, C)], ebuf, sem_e)
        cp_k.wait()
        cp_qv.wait()
        cp_e.wait()

        def edge(i, _):
            for j in range(D // L):
                sl = pl.ds(j * L, L)
                kv = kbuf[i, sl]
                qv = qvbuf[i, sl]
                vv = qvbuf[i, pl.ds(D + j * L, L)]
                ev = ebuf[i, sl]
                z = kv + qv + 2.0 * ev
                g = 1.0 / (1.0 + jnp.exp(-z))
                msgbuf[i, sl] = g * (vv + ev)
            return 0

        lax.fori_loop(0, C, edge, 0)
        pltpu.sync_copy(msgbuf, agg_sh.at[idx_d], add=True)
        return 0

    lax.fori_loop(0, NCHUNK, chunk, 0)

    plsc.subcore_barrier()
    pltpu.sync_copy(agg_sh.at[pl.ds(s * RPT, RPT)],
                    out_hbm.at[pl.ds((c * N) + s * RPT, RPT)])


_sc_edge = pl.kernel(
    _sc_edge_body,
    out_type=jax.ShapeDtypeStruct((NC * N, D), jnp.float32),
    mesh=plsc.VectorSubcoreMesh(core_axis_name="c", subcore_axis_name="s",
                                num_cores=NC, num_subcores=NS),
    scratch_types=[
        pltpu.VMEM((C,), jnp.int32),
        pltpu.VMEM((C,), jnp.int32),
        pltpu.VMEM((C, D), jnp.float32),
        pltpu.VMEM((C, 2 * D), jnp.float32),
        pltpu.VMEM((C, D), jnp.float32),
        pltpu.VMEM((C, D), jnp.float32),
        pltpu.VMEM_SHARED((N, D), jnp.float32),
        pltpu.SemaphoreType.DMA,
        pltpu.SemaphoreType.DMA,
        pltpu.SemaphoreType.DMA,
    ],
)


def kernel(x, edge_index, edge_attr, params):
    p = params
    src = edge_index[0]
    dst = edge_index[1]
    zeros = jnp.zeros((RPT, D), jnp.float32)

    def b2(v):
        return v.reshape(1, D)

    e1, e2, e3 = _edge_proj(edge_attr, p["We1"], p["We2"], p["We3"])

    k1, qv1, s1 = _node_proj(
        x, p["Wk1"], p["Wq1"], p["Wv1"], p["Wskip1"],
        b2(p["bk1"]), b2(p["bq1"]), b2(p["bv1"]), b2(p["b1"]))
    agg1 = _sc_edge(k1, qv1, e1, src, dst, zeros).reshape(NC, N, D)

    _, k2, qv2, s2 = _fused_proj(
        agg1, s1, p["Wk2"], p["Wq2"], p["Wv2"], p["Wskip2"],
        b2(p["bk2"]), b2(p["bq2"]), b2(p["bv2"]), b2(p["b2"]))
    agg2 = _sc_edge(k2, qv2, e2, src, dst, zeros).reshape(NC, N, D)

    feat, k3, qv3, s3 = _fused_proj(
        agg2, s2, p["Wk3"], p["Wq3"], p["Wv3"], p["Wskip3"],
        b2(p["bk3"]), b2(p["bq3"]), b2(p["bv3"]), b2(p["b3"]))
    agg3 = _sc_edge(k3, qv3, e3, src, dst, zeros).reshape(NC, N, D)

    wl_pad = jnp.zeros((D, D), jnp.float32).at[:, 0].set(p["Wlin"][:, 0])
    bl_pad = jnp.zeros((1, D), jnp.float32).at[0, 0].set(p["blin"][0])
    lin_full, sig_full = _final(agg3, s3, wl_pad, bl_pad)
    lin = lin_full[:, :1]
    sig = sig_full[:, :1]

    return (sig, feat, jnp.concatenate([feat, lin], axis=1))


# speedup vs baseline: 4.4959x; 4.4959x over previous
import jax
import jax.numpy as jnp
from jax import lax
from jax.experimental import pallas as pl
from jax.experimental.pallas import tpu as pltpu
from jax.experimental.pallas import tpu_sc as plsc

N = 10000
E = 320000
D = 128
D_EDGE = 16

NC, NS, L = 2, 16, 16
NW = NC * NS
EPW = E // NW
C = 40
NCHUNK = EPW // C
RPT = 624
REM = N - RPT * NS

BM = 2000
BE = 4000


def _edge_proj_body(a_ref, w1_ref, w2_ref, w3_ref, o1_ref, o2_ref, o3_ref):
    a = a_ref[...]
    o1_ref[...] = jnp.dot(a, w1_ref[...], preferred_element_type=jnp.float32)
    o2_ref[...] = jnp.dot(a, w2_ref[...], preferred_element_type=jnp.float32)
    o3_ref[...] = jnp.dot(a, w3_ref[...], preferred_element_type=jnp.float32)


def _edge_proj(edge_attr, w1, w2, w3):
    wspec = pl.BlockSpec((D_EDGE, D), lambda i: (0, 0))
    ospec = pl.BlockSpec((BE, D), lambda i: (i, 0))
    return pl.pallas_call(
        _edge_proj_body,
        grid=(E // BE,),
        in_specs=[pl.BlockSpec((BE, D_EDGE), lambda i: (i, 0)), wspec, wspec, wspec],
        out_specs=[ospec, ospec, ospec],
        out_shape=[jax.ShapeDtypeStruct((E, D), jnp.float32)] * 3,
    )(edge_attr, w1, w2, w3)


def _pack_qv(q, v):
    qb = jax.lax.bitcast_convert_type(q.astype(jnp.bfloat16), jnp.uint16)
    vb = jax.lax.bitcast_convert_type(v.astype(jnp.bfloat16), jnp.uint16)
    return qb.astype(jnp.uint32) | (vb.astype(jnp.uint32) << 16)


def _proj_common(h, wk_ref, wq_ref, wv_ref, ws_ref, bk_ref, bq_ref, bv_ref,
                 bb_ref, k_ref, qv_ref, s_ref):
    k_ref[...] = jnp.dot(h, wk_ref[...], preferred_element_type=jnp.float32) + bk_ref[...]
    q = jnp.dot(h, wq_ref[...], preferred_element_type=jnp.float32) + bq_ref[...]
    v = jnp.dot(h, wv_ref[...], preferred_element_type=jnp.float32) + bv_ref[...]
    qv_ref[...] = _pack_qv(q, v)
    s_ref[...] = jnp.dot(h, ws_ref[...], preferred_element_type=jnp.float32) + bb_ref[...]


def _proj_body(h_ref, wk_ref, wq_ref, wv_ref, ws_ref,
               bk_ref, bq_ref, bv_ref, bb_ref,
               k_ref, qv_ref, s_ref):
    _proj_common(h_ref[...], wk_ref, wq_ref, wv_ref, ws_ref,
                 bk_ref, bq_ref, bv_ref, bb_ref, k_ref, qv_ref, s_ref)


def _fused_proj_body(agg_ref, sp_ref, wk_ref, wq_ref, wv_ref, ws_ref,
                     bk_ref, bq_ref, bv_ref, bb_ref,
                     h_ref, k_ref, qv_ref, s_ref):
    z = agg_ref[0] + agg_ref[1] + sp_ref[...]
    h = jnp.where(z > 0, z, jnp.exp(jnp.minimum(z, 0.0)) - 1.0)
    h_ref[...] = h
    _proj_common(h, wk_ref, wq_ref, wv_ref, ws_ref,
                 bk_ref, bq_ref, bv_ref, bb_ref, k_ref, qv_ref, s_ref)


_WSPEC = pl.BlockSpec((D, D), lambda i: (0, 0))
_BSPEC = pl.BlockSpec((1, D), lambda i: (0, 0))
_HSPEC = pl.BlockSpec((BM, D), lambda i: (i, 0))
_AGGSPEC = pl.BlockSpec((2, BM, D), lambda i: (0, i, 0))

_NODE_OUT = [
    jax.ShapeDtypeStruct((N, D), jnp.float32),
    jax.ShapeDtypeStruct((N, D), jnp.uint32),
    jax.ShapeDtypeStruct((N, D), jnp.float32),
]


def _node_proj(x, wk, wq, wv, ws, bk, bq, bv, bb):
    return pl.pallas_call(
        _proj_body,
        grid=(N // BM,),
        in_specs=[_HSPEC] + [_WSPEC] * 4 + [_BSPEC] * 4,
        out_specs=[_HSPEC, _HSPEC, _HSPEC],
        out_shape=_NODE_OUT,
    )(x, wk, wq, wv, ws, bk, bq, bv, bb)


def _fused_proj(agg, sp, wk, wq, wv, ws, bk, bq, bv, bb):
    return pl.pallas_call(
        _fused_proj_body,
        grid=(N // BM,),
        in_specs=[_AGGSPEC, _HSPEC] + [_WSPEC] * 4 + [_BSPEC] * 4,
        out_specs=[_HSPEC, _HSPEC, _HSPEC, _HSPEC],
        out_shape=[jax.ShapeDtypeStruct((N, D), jnp.float32)] + _NODE_OUT,
    )(agg, sp, wk, wq, wv, ws, bk, bq, bv, bb)


def _final_body(agg_ref, sp_ref, wl_ref, bl_ref, lin_ref, sig_ref):
    z = agg_ref[0] + agg_ref[1] + sp_ref[...]
    h = jnp.where(z > 0, z, jnp.exp(jnp.minimum(z, 0.0)) - 1.0)
    lin = jnp.dot(h, wl_ref[...], preferred_element_type=jnp.float32) + bl_ref[...]
    lin_ref[...] = lin
    sig_ref[...] = jax.nn.sigmoid(lin)


def _final(agg, sp, wl_pad, bl_pad):
    return pl.pallas_call(
        _final_body,
        grid=(N // BM,),
        in_specs=[_AGGSPEC, _HSPEC, _WSPEC, _BSPEC],
        out_specs=[_HSPEC, _HSPEC],
        out_shape=[jax.ShapeDtypeStruct((N, D), jnp.float32)] * 2,
    )(agg, sp, wl_pad, bl_pad)


def _sc_edge_body(k_hbm, qv_hbm, e_hbm, src_hbm, dst_hbm, z_hbm, out_hbm,
                  is0, id0, is1, id1, is2, id2, is3, id3,
                  kb0, qb0, eb0, kb1, qb1, eb1,
                  msg0, msg1, agg_sh, semi0, semi1, semi2, semi3,
                  semg0, semg1, sems0, sems1):
    c = lax.axis_index("c")
    s = lax.axis_index("s")
    wid = c * NS + s
    ebase = wid * EPW

    isl = (is0, is1, is2, is3)
    idl = (id0, id1, id2, id3)
    kb = (kb0, kb1)
    qb = (qb0, qb1)
    eb = (eb0, eb1)
    msgl = (msg0, msg1)
    semi = (semi0, semi1, semi2, semi3)
    semg = (semg0, semg1)
    sems = (sems0, sems1)

    pltpu.sync_copy(z_hbm.at[pl.ds(0, RPT)], agg_sh.at[pl.ds(s * RPT, RPT)])

    @pl.when(s == NS - 1)
    def _():
        pltpu.sync_copy(z_hbm.at[pl.ds(0, REM)], agg_sh.at[pl.ds(RPT * NS, REM)])

    plsc.subcore_barrier()

    def issue_idx(ci, b):
        base = ebase + ci * C
        pltpu.async_copy(src_hbm.at[pl.ds(base, C)], isl[b], semi[b])
        pltpu.async_copy(dst_hbm.at[pl.ds(base, C)], idl[b], semi[b])

    def wait_idx(b):
        pltpu.make_async_copy(src_hbm.at[pl.ds(0, C)], isl[b], semi[b]).wait()
        pltpu.make_async_copy(dst_hbm.at[pl.ds(0, C)], idl[b], semi[b]).wait()

    def issue_gathers(ci, i4, d):
        base = ebase + ci * C
        pltpu.async_copy(k_hbm.at[idl[i4]], kb[d], semg[d])
        pltpu.async_copy(qv_hbm.at[isl[i4]], qb[d], semg[d])
        pltpu.async_copy(e_hbm.at[pl.ds(base, C)], eb[d], semg[d])

    def wait_gathers(i4, d):
        pltpu.make_async_copy(k_hbm.at[idl[i4]], kb[d], semg[d]).wait()
        pltpu.make_async_copy(qv_hbm.at[isl[i4]], qb[d], semg[d]).wait()
        pltpu.make_async_copy(e_hbm.at[pl.ds(0, C)], eb[d], semg[d]).wait()

    def wait_scatter(i4, d):
        pltpu.make_async_copy(msgl[d], agg_sh.at[idl[i4]], sems[d]).wait()

    def compute(b):
        kbuf, qvbuf, ebuf, msgb = kb[b], qb[b], eb[b], msgl[b]

        @plsc.parallel_loop(0, C, 1, unroll=2)
        def edge(i):
            qvw = [qvbuf[i, pl.ds(j * L, L)] for j in range(D // L)]
            qs = [jax.lax.bitcast_convert_type(w << 16, jnp.float32)
                  for w in qvw]
            vs = [jax.lax.bitcast_convert_type(w & jnp.uint32(0xFFFF0000),
                                               jnp.float32) for w in qvw]
            evs = [ebuf[i, pl.ds(j * L, L)] for j in range(D // L)]
            zs = [kbuf[i, pl.ds(j * L, L)] + qs[j] + 2.0 * evs[j]
                  for j in range(D // L)]
            gs = [1.0 / (1.0 + jnp.exp(-z)) for z in zs]
            for j in range(D // L):
                msgb[i, pl.ds(j * L, L)] = gs[j] * (vs[j] + evs[j])

    issue_idx(0, 0)
    wait_idx(0)
    issue_gathers(0, 0, 0)
    issue_idx(1, 1)

    def pair(g, _):
        for b in range(2):
            cur = 2 * g + b
            nb = 1 - b
            wait_gathers(b, b)
            compute(b)

            @pl.when(cur + 1 < NCHUNK)
            def _():
                wait_idx(nb)
                issue_gathers(cur + 1, nb, nb)

            pltpu.sync_copy(msgl[b], agg_sh.at[idl[b]], add=True)

            @pl.when(cur + 2 < NCHUNK)
            def _():
                issue_idx(cur + 2, b)
        return 0

    lax.fori_loop(0, NCHUNK // 2, pair, 0)

    plsc.subcore_barrier()
    pltpu.sync_copy(agg_sh.at[pl.ds(s * RPT, RPT)],
                    out_hbm.at[pl.ds((c * N) + s * RPT, RPT)])

    @pl.when(s == NS - 1)
    def _():
        pltpu.sync_copy(agg_sh.at[pl.ds(RPT * NS, REM)],
                        out_hbm.at[pl.ds((c * N) + RPT * NS, REM)])


_sc_edge = pl.kernel(
    _sc_edge_body,
    out_type=jax.ShapeDtypeStruct((NC * N, D), jnp.float32),
    mesh=plsc.VectorSubcoreMesh(core_axis_name="c", subcore_axis_name="s",
                                num_cores=NC, num_subcores=NS),
    scratch_types=(
        [pltpu.VMEM((C,), jnp.int32)] * 8
        + [
            pltpu.VMEM((C, D), jnp.float32),
            pltpu.VMEM((C, D), jnp.uint32),
            pltpu.VMEM((C, D), jnp.float32),
            pltpu.VMEM((C, D), jnp.float32),
            pltpu.VMEM((C, D), jnp.uint32),
            pltpu.VMEM((C, D), jnp.float32),
            pltpu.VMEM((C, D), jnp.float32),
            pltpu.VMEM((C, D), jnp.float32),
            pltpu.VMEM_SHARED((N, D), jnp.float32),
        ]
        + [pltpu.SemaphoreType.DMA] * 8
    ),
)


def kernel(x, edge_index, edge_attr, params):
    p = params
    src = edge_index[0]
    dst = edge_index[1]
    zeros = jnp.zeros((RPT, D), jnp.float32)

    def b2(v):
        return v.reshape(1, D)

    e1, e2, e3 = _edge_proj(edge_attr, p["We1"], p["We2"], p["We3"])

    k1, qv1, s1 = _node_proj(
        x, p["Wk1"], p["Wq1"], p["Wv1"], p["Wskip1"],
        b2(p["bk1"]), b2(p["bq1"]), b2(p["bv1"]), b2(p["b1"]))
    agg1 = _sc_edge(k1, qv1, e1, src, dst, zeros).reshape(NC, N, D)

    _, k2, qv2, s2 = _fused_proj(
        agg1, s1, p["Wk2"], p["Wq2"], p["Wv2"], p["Wskip2"],
        b2(p["bk2"]), b2(p["bq2"]), b2(p["bv2"]), b2(p["b2"]))
    agg2 = _sc_edge(k2, qv2, e2, src, dst, zeros).reshape(NC, N, D)

    feat, k3, qv3, s3 = _fused_proj(
        agg2, s2, p["Wk3"], p["Wq3"], p["Wv3"], p["Wskip3"],
        b2(p["bk3"]), b2(p["bq3"]), b2(p["bv3"]), b2(p["b3"]))
    agg3 = _sc_edge(k3, qv3, e3, src, dst, zeros).reshape(NC, N, D)

    wl_pad = jnp.zeros((D, D), jnp.float32).at[:, 0].set(p["Wlin"][:, 0])
    bl_pad = jnp.zeros((1, D), jnp.float32).at[0, 0].set(p["blin"][0])
    lin_full, sig_full = _final(agg3, s3, wl_pad, bl_pad)
    lin = lin_full[:, :1]
    sig = sig_full[:, :1]

    return (sig, feat, jnp.concatenate([feat, lin], axis=1))
